# baseline (speedup 1.0000x reference)
import functools

import jax
import jax.numpy as jnp
from jax import lax
from jax.experimental import pallas as pl
from jax.experimental.pallas import tpu as pltpu
from jax.experimental.pallas import tpu_sc as plsc

B, L, F, N = 16, 4096, 128, 512
TC_B = 2
SC_B = B - TC_B
NC, NS = 2, 16
BPC = SC_B // NC
ROWS_PER_TILE = BPC * L // NS
CHUNK = 128
NCHUNK = ROWS_PER_TILE // CHUNK
ACC_ROWS = BPC * N
SHARE = ACC_ROWS // NS
NBUF = 5
LOOKAHEAD = 3

_mesh = plsc.VectorSubcoreMesh(core_axis_name="c", subcore_axis_name="s")


@functools.partial(
    pl.kernel,
    out_type=jax.ShapeDtypeStruct((B * N, F), jnp.float32),
    mesh=_mesh,
    scratch_types=[
        pltpu.VMEM((NCHUNK, 1, CHUNK), jnp.int32),
        [pltpu.VMEM((CHUNK, F), jnp.float32) for _ in range(NBUF)],
        pltpu.VMEM((32, F), jnp.float32),
        pltpu.VMEM_SHARED((ACC_ROWS, F), jnp.float32),
        [pltpu.SemaphoreType.DMA for _ in range(NBUF)],
        [pltpu.SemaphoreType.DMA for _ in range(NBUF)],
        pltpu.SemaphoreType.DMA,
        pltpu.SemaphoreType.DMA,
    ],
)
def _segsum_sc(data_hbm, seg_hbm, out_hbm,
               idx_v, bufs, zbuf, acc_sh, gsems, ssems, isem, segsem):
    cid = lax.axis_index("c")
    sid = lax.axis_index("s")

    tile_row0 = cid * BPC * L + sid * ROWS_PER_TILE

    zero = jnp.zeros((16,), jnp.float32)
    for r in range(32):
        for j in range(F // 16):
            zbuf[r, pl.ds(j * 16, 16)] = zero
    init = [
        pltpu.async_copy(
            zbuf, acc_sh.at[pl.ds(sid * SHARE + k * 32, 32)], isem
        )
        for k in range(SHARE // 32)
    ]

    def gather(i, b):
        return pltpu.async_copy(
            data_hbm.at[pl.ds(tile_row0 + i * CHUNK, CHUNK)], bufs[b], gsems[b]
        )

    gd = [None] * NBUF
    sd = [None] * NBUF
    for i in range(LOOKAHEAD):
        gd[i] = gather(i, i)

    segd = []
    for i in range(NCHUNK):
        r = tile_row0 + i * CHUNK
        bb = r // L
        segd.append(pltpu.async_copy(
            seg_hbm.at[bb, pl.ds(r - bb * L, CHUNK)], idx_v.at[i, 0], segsem
        ))
    for d in segd:
        d.wait()
    for i in range(NCHUNK):
        r = tile_row0 + i * CHUNK
        seg_off = (r // L - cid * BPC) * N
        for j in range(CHUNK // 16):
            sl = pl.ds(j * 16, 16)
            idx_v[i, 0, sl] = idx_v[i, 0, sl] + seg_off

    for d in init:
        d.wait()
    plsc.subcore_barrier()

    for i in range(NCHUNK):
        bc = i % NBUF
        nxt = i + LOOKAHEAD
        if nxt < NCHUNK:
            bn = nxt % NBUF
            if sd[bn] is not None:
                sd[bn].wait()
                sd[bn] = None
            gd[bn] = gather(nxt, bn)
        gd[bc].wait()
        sd[bc] = pltpu.async_copy(
            bufs[bc], acc_sh.at[idx_v.at[i, 0]], ssems[bc], add=True
        )

    for b in range(NBUF):
        if sd[b] is not None:
            sd[b].wait()

    plsc.subcore_barrier()
    pltpu.sync_copy(
        acc_sh.at[pl.ds(sid * SHARE, SHARE)],
        out_hbm.at[pl.ds(cid * ACC_ROWS + sid * SHARE, SHARE)],
    )


@functools.partial(
    pl.pallas_call,
    grid=(TC_B,),
    in_specs=[
        pl.BlockSpec((1, L, F), lambda b: (b + SC_B, 0, 0)),
        pl.BlockSpec((8, L), lambda b: (1, 0)),
    ],
    out_specs=pl.BlockSpec((1, N, F), lambda b: (b, 0, 0)),
    out_shape=jax.ShapeDtypeStruct((TC_B, N, F), jnp.float32),
)
def _segsum_tc(data_ref, seg_ref, out_ref):
    b = pl.program_id(0)
    seg = seg_ref[b + SC_B - 8, :]
    oht = (lax.broadcasted_iota(jnp.int32, (N, L), 0) == seg[None, :])
    out_ref[0] = lax.dot_general(
        oht.astype(jnp.bfloat16),
        data_ref[0].astype(jnp.bfloat16),
        (((1,), (0,)), ((), ())),
        preferred_element_type=jnp.float32,
    )


def kernel(data, segment_ids, target):
    flat_data = data.reshape(B * L, F)
    sc_out = _segsum_sc(flat_data, segment_ids)
    tc_out = _segsum_tc(data, segment_ids)
    full = sc_out.reshape(B, N, F)
    return lax.dynamic_update_slice(full, tc_out, (SC_B, 0, 0))


# speedup vs baseline: 1.0788x; 1.0788x over previous
import functools

import jax
import jax.numpy as jnp
from jax import lax
from jax.experimental import pallas as pl
from jax.experimental.pallas import tpu as pltpu
from jax.experimental.pallas import tpu_sc as plsc

B, L, F, N = 16, 4096, 128, 512
TC_B = 6
SC_B = B - TC_B
NC, NS = 2, 16
BPC = SC_B // NC
ROWS_PER_TILE = BPC * L // NS
CHUNK = 128
NCHUNK = ROWS_PER_TILE // CHUNK
ACC_ROWS = BPC * N
SHARE = ACC_ROWS // NS
NBUF = 5
LOOKAHEAD = 3

_mesh = plsc.VectorSubcoreMesh(core_axis_name="c", subcore_axis_name="s")


@functools.partial(
    pl.kernel,
    out_type=jax.ShapeDtypeStruct((B * N, F), jnp.float32),
    mesh=_mesh,
    scratch_types=[
        pltpu.VMEM((NCHUNK, 1, CHUNK), jnp.int32),
        [pltpu.VMEM((CHUNK, F), jnp.float32) for _ in range(NBUF)],
        pltpu.VMEM((32, F), jnp.float32),
        pltpu.VMEM_SHARED((ACC_ROWS, F), jnp.float32),
        [pltpu.SemaphoreType.DMA for _ in range(NBUF)],
        [pltpu.SemaphoreType.DMA for _ in range(NBUF)],
        pltpu.SemaphoreType.DMA,
        pltpu.SemaphoreType.DMA,
    ],
)
def _segsum_sc(data_hbm, seg_hbm, out_hbm,
               idx_v, bufs, zbuf, acc_sh, gsems, ssems, isem, segsem):
    cid = lax.axis_index("c")
    sid = lax.axis_index("s")

    tile_row0 = cid * BPC * L + sid * ROWS_PER_TILE

    zero = jnp.zeros((16,), jnp.float32)
    for r in range(32):
        for j in range(F // 16):
            zbuf[r, pl.ds(j * 16, 16)] = zero
    init = [
        pltpu.async_copy(
            zbuf, acc_sh.at[pl.ds(sid * SHARE + k * 32, 32)], isem
        )
        for k in range(SHARE // 32)
    ]

    def gather(i, b):
        return pltpu.async_copy(
            data_hbm.at[pl.ds(tile_row0 + i * CHUNK, CHUNK)], bufs[b], gsems[b]
        )

    gd = [None] * NBUF
    sd = [None] * NBUF
    for i in range(LOOKAHEAD):
        gd[i] = gather(i, i)

    segd = []
    for i in range(NCHUNK):
        r = tile_row0 + i * CHUNK
        bb = r // L
        segd.append(pltpu.async_copy(
            seg_hbm.at[bb, pl.ds(r - bb * L, CHUNK)], idx_v.at[i, 0], segsem
        ))
    for d in segd:
        d.wait()
    for i in range(NCHUNK):
        r = tile_row0 + i * CHUNK
        seg_off = (r // L - cid * BPC) * N
        for j in range(CHUNK // 16):
            sl = pl.ds(j * 16, 16)
            idx_v[i, 0, sl] = idx_v[i, 0, sl] + seg_off

    for d in init:
        d.wait()
    plsc.subcore_barrier()

    for i in range(NCHUNK):
        bc = i % NBUF
        nxt = i + LOOKAHEAD
        if nxt < NCHUNK:
            bn = nxt % NBUF
            if sd[bn] is not None:
                sd[bn].wait()
                sd[bn] = None
            gd[bn] = gather(nxt, bn)
        gd[bc].wait()
        sd[bc] = pltpu.async_copy(
            bufs[bc], acc_sh.at[idx_v.at[i, 0]], ssems[bc], add=True
        )

    for b in range(NBUF):
        if sd[b] is not None:
            sd[b].wait()

    plsc.subcore_barrier()
    pltpu.sync_copy(
        acc_sh.at[pl.ds(sid * SHARE, SHARE)],
        out_hbm.at[pl.ds(cid * ACC_ROWS + sid * SHARE, SHARE)],
    )


@functools.partial(
    pl.pallas_call,
    grid=(TC_B,),
    in_specs=[
        pl.BlockSpec((1, L, F), lambda b: (b + SC_B, 0, 0)),
        pl.BlockSpec((8, L), lambda b: (1, 0)),
    ],
    out_specs=pl.BlockSpec((1, N, F), lambda b: (b, 0, 0)),
    out_shape=jax.ShapeDtypeStruct((TC_B, N, F), jnp.float32),
)
def _segsum_tc(data_ref, seg_ref, out_ref):
    b = pl.program_id(0)
    seg = seg_ref[b + SC_B - 8, :]
    oht = (lax.broadcasted_iota(jnp.int32, (N, L), 0) == seg[None, :])
    out_ref[0] = lax.dot_general(
        oht.astype(jnp.bfloat16),
        data_ref[0].astype(jnp.bfloat16),
        (((1,), (0,)), ((), ())),
        preferred_element_type=jnp.float32,
    )


def kernel(data, segment_ids, target):
    flat_data = data.reshape(B * L, F)
    sc_out = _segsum_sc(flat_data, segment_ids)
    tc_out = _segsum_tc(data, segment_ids)
    full = sc_out.reshape(B, N, F)
    return lax.dynamic_update_slice(full, tc_out, (SC_B, 0, 0))


# speedup vs baseline: 1.1407x; 1.0574x over previous
import functools

import jax
import jax.numpy as jnp
from jax import lax
from jax.experimental import pallas as pl
from jax.experimental.pallas import tpu as pltpu
from jax.experimental.pallas import tpu_sc as plsc

B, L, F, N = 16, 4096, 128, 512
TC_B = 8
SC_B = B - TC_B
NC, NS = 2, 16
BPC = SC_B // NC
ROWS_PER_TILE = BPC * L // NS
CHUNK = 128
NCHUNK = ROWS_PER_TILE // CHUNK
ACC_ROWS = BPC * N
SHARE = ACC_ROWS // NS
NBUF = 5
LOOKAHEAD = 3

_mesh = plsc.VectorSubcoreMesh(core_axis_name="c", subcore_axis_name="s")


@functools.partial(
    pl.kernel,
    out_type=jax.ShapeDtypeStruct((B * N, F), jnp.float32),
    mesh=_mesh,
    scratch_types=[
        pltpu.VMEM((NCHUNK, 1, CHUNK), jnp.int32),
        [pltpu.VMEM((CHUNK, F), jnp.float32) for _ in range(NBUF)],
        pltpu.VMEM((32, F), jnp.float32),
        pltpu.VMEM_SHARED((ACC_ROWS, F), jnp.float32),
        [pltpu.SemaphoreType.DMA for _ in range(NBUF)],
        [pltpu.SemaphoreType.DMA for _ in range(NBUF)],
        pltpu.SemaphoreType.DMA,
        pltpu.SemaphoreType.DMA,
    ],
)
def _segsum_sc(data_hbm, seg_hbm, out_hbm,
               idx_v, bufs, zbuf, acc_sh, gsems, ssems, isem, segsem):
    cid = lax.axis_index("c")
    sid = lax.axis_index("s")

    tile_row0 = cid * BPC * L + sid * ROWS_PER_TILE

    zero = jnp.zeros((16,), jnp.float32)
    for r in range(32):
        for j in range(F // 16):
            zbuf[r, pl.ds(j * 16, 16)] = zero
    init = [
        pltpu.async_copy(
            zbuf, acc_sh.at[pl.ds(sid * SHARE + k * 32, 32)], isem
        )
        for k in range(SHARE // 32)
    ]

    def gather(i, b):
        return pltpu.async_copy(
            data_hbm.at[pl.ds(tile_row0 + i * CHUNK, CHUNK)], bufs[b], gsems[b]
        )

    gd = [None] * NBUF
    sd = [None] * NBUF
    for i in range(LOOKAHEAD):
        gd[i] = gather(i, i)

    segd = []
    for i in range(NCHUNK):
        r = tile_row0 + i * CHUNK
        bb = r // L
        segd.append(pltpu.async_copy(
            seg_hbm.at[bb, pl.ds(r - bb * L, CHUNK)], idx_v.at[i, 0], segsem
        ))
    for d in segd:
        d.wait()
    for i in range(NCHUNK):
        r = tile_row0 + i * CHUNK
        seg_off = (r // L - cid * BPC) * N
        for j in range(CHUNK // 16):
            sl = pl.ds(j * 16, 16)
            idx_v[i, 0, sl] = idx_v[i, 0, sl] + seg_off

    for d in init:
        d.wait()
    plsc.subcore_barrier()

    for i in range(NCHUNK):
        bc = i % NBUF
        nxt = i + LOOKAHEAD
        if nxt < NCHUNK:
            bn = nxt % NBUF
            if sd[bn] is not None:
                sd[bn].wait()
                sd[bn] = None
            gd[bn] = gather(nxt, bn)
        gd[bc].wait()
        sd[bc] = pltpu.async_copy(
            bufs[bc], acc_sh.at[idx_v.at[i, 0]], ssems[bc], add=True
        )

    for b in range(NBUF):
        if sd[b] is not None:
            sd[b].wait()

    plsc.subcore_barrier()
    pltpu.sync_copy(
        acc_sh.at[pl.ds(sid * SHARE, SHARE)],
        out_hbm.at[pl.ds(cid * ACC_ROWS + sid * SHARE, SHARE)],
    )


@functools.partial(
    pl.pallas_call,
    grid=(TC_B,),
    in_specs=[
        pl.BlockSpec((1, L, F), lambda b: (b + SC_B, 0, 0)),
        pl.BlockSpec((8, L), lambda b: (1, 0)),
    ],
    out_specs=pl.BlockSpec((1, N, F), lambda b: (b, 0, 0)),
    out_shape=jax.ShapeDtypeStruct((TC_B, N, F), jnp.float32),
)
def _segsum_tc(data_ref, seg_ref, out_ref):
    b = pl.program_id(0)
    seg = seg_ref[b + SC_B - 8, :]
    oht = (lax.broadcasted_iota(jnp.int32, (N, L), 0) == seg[None, :])
    out_ref[0] = lax.dot_general(
        oht.astype(jnp.bfloat16),
        data_ref[0].astype(jnp.bfloat16),
        (((1,), (0,)), ((), ())),
        preferred_element_type=jnp.float32,
    )


def kernel(data, segment_ids, target):
    flat_data = data.reshape(B * L, F)
    sc_out = _segsum_sc(flat_data, segment_ids)
    tc_out = _segsum_tc(data, segment_ids)
    full = sc_out.reshape(B, N, F)
    return lax.dynamic_update_slice(full, tc_out, (SC_B, 0, 0))
